# baseline (speedup 1.0000x reference)
import functools

import jax
import jax.numpy as jnp
from jax import lax
from jax.experimental import pallas as pl
from jax.experimental.pallas import tpu as pltpu
from jax.experimental.pallas import tpu_sc as plsc

_LANES = 16


@functools.lru_cache(maxsize=None)
def _build_gather(B, N, K, D, KP):
    info = plsc.get_sparse_core_info()
    NC, NS = info.num_cores, info.num_subcores
    NW = NC * NS
    ND = 32
    assert D % ND == 0
    DW = D // ND
    ntask = K * ND
    assert ntask % NW == 0
    tpw = ntask // NW
    NB = 12
    SKEW = 6

    mesh = plsc.VectorSubcoreMesh(core_axis_name="c", subcore_axis_name="s")

    @functools.partial(
        pl.kernel,
        mesh=mesh,
        compiler_params=pltpu.CompilerParams(
            use_tc_tiling_on_sc=True, needs_layout_passes=False
        ),
        out_type=jax.ShapeDtypeStruct((K, D, B), jnp.float32),
        scratch_types=[
            pltpu.VMEM((KP,), jnp.int32),
            pltpu.VMEM((NB, 1, DW, B), jnp.float32),
            pltpu.SemaphoreType.DMA((NB,)),
            pltpu.SemaphoreType.DMA((NB,)),
        ],
    )
    def gather_kernel(tt_hbm, idx_hbm, out_hbm, idx_v, bufs, gsem, wsem):
        wid = lax.axis_index("s") * NC + lax.axis_index("c")
        pltpu.sync_copy(idx_hbm, idx_v.at[pl.ds(0, K)])
        lane = lax.iota(jnp.int32, _LANES)

        reads = [None] * tpw
        writes = [None] * tpw
        dsts = [None] * tpw
        for s in range(tpw + SKEW):
            if s < tpw:
                slot = s % NB
                if s >= NB:
                    writes[s - NB].wait()
                g = wid * tpw + s
                j = lax.div(g, ND)
                dp = g - j * ND
                c = lax.div(j, _LANES)
                vec = idx_v[pl.ds(c * _LANES, _LANES)]
                ij = jnp.sum(jnp.where(lane == (j - c * _LANES), vec, 0))
                dsts[s] = (j, dp)
                reads[s] = pltpu.async_copy(
                    tt_hbm.at[pl.ds(ij, 1), pl.ds(dp * DW, DW)],
                    bufs.at[slot],
                    gsem.at[slot],
                )
            w = s - SKEW
            if w >= 0:
                reads[w].wait()
                j, dp = dsts[w]
                writes[w] = pltpu.async_copy(
                    bufs.at[w % NB],
                    out_hbm.at[pl.ds(j, 1), pl.ds(dp * DW, DW)],
                    wsem.at[w % NB],
                )
        for w in range(max(0, tpw - NB), tpw):
            writes[w].wait()

    return gather_kernel


def kernel(tensor, indices):
    B, N, D = tensor.shape
    K = indices.shape[0]
    KP = (K + _LANES - 1) // _LANES * _LANES
    idx32 = indices.astype(jnp.int32)
    tt = jnp.transpose(tensor, (1, 2, 0))
    out_t = _build_gather(B, N, K, D, KP)(tt, idx32)
    return jnp.transpose(out_t, (2, 0, 1))


# speedup vs baseline: 1.0363x; 1.0363x over previous
import functools

import jax
import jax.numpy as jnp
from jax import lax
from jax.experimental import pallas as pl
from jax.experimental.pallas import tpu as pltpu
from jax.experimental.pallas import tpu_sc as plsc

_LANES = 16


@functools.lru_cache(maxsize=None)
def _build_gather(B, N, K, D, KP):
    info = plsc.get_sparse_core_info()
    NC, NS = info.num_cores, info.num_subcores
    NW = NC * NS
    ND = 16
    assert D % ND == 0
    DW = D // ND
    ntask = K * ND
    assert ntask % NW == 0
    tpw = ntask // NW
    NB = 6
    SKEW = 3

    mesh = plsc.VectorSubcoreMesh(core_axis_name="c", subcore_axis_name="s")

    @functools.partial(
        pl.kernel,
        mesh=mesh,
        compiler_params=pltpu.CompilerParams(
            use_tc_tiling_on_sc=True, needs_layout_passes=False
        ),
        out_type=jax.ShapeDtypeStruct((K, D, B), jnp.float32),
        scratch_types=[
            pltpu.VMEM((KP,), jnp.int32),
            pltpu.VMEM((NB, 1, DW, B), jnp.float32),
            pltpu.SemaphoreType.DMA((NB,)),
            pltpu.SemaphoreType.DMA((NB,)),
        ],
    )
    def gather_kernel(tt_hbm, idx_hbm, out_hbm, idx_v, bufs, gsem, wsem):
        wid = lax.axis_index("s") * NC + lax.axis_index("c")
        pltpu.sync_copy(idx_hbm, idx_v.at[pl.ds(0, K)])
        lane = lax.iota(jnp.int32, _LANES)

        reads = [None] * tpw
        writes = [None] * tpw
        dsts = [None] * tpw
        for s in range(tpw + SKEW):
            if s < tpw:
                slot = s % NB
                if s >= NB:
                    writes[s - NB].wait()
                g = wid * tpw + s
                j = lax.div(g, ND)
                dp = g - j * ND
                c = lax.div(j, _LANES)
                vec = idx_v[pl.ds(c * _LANES, _LANES)]
                ij = jnp.sum(jnp.where(lane == (j - c * _LANES), vec, 0))
                dsts[s] = (j, dp)
                reads[s] = pltpu.async_copy(
                    tt_hbm.at[pl.ds(ij, 1), pl.ds(dp * DW, DW)],
                    bufs.at[slot],
                    gsem.at[slot],
                )
            w = s - SKEW
            if w >= 0:
                reads[w].wait()
                j, dp = dsts[w]
                writes[w] = pltpu.async_copy(
                    bufs.at[w % NB],
                    out_hbm.at[pl.ds(j, 1), pl.ds(dp * DW, DW)],
                    wsem.at[w % NB],
                )
        for w in range(max(0, tpw - NB), tpw):
            writes[w].wait()

    return gather_kernel


def kernel(tensor, indices):
    B, N, D = tensor.shape
    K = indices.shape[0]
    KP = (K + _LANES - 1) // _LANES * _LANES
    idx32 = indices.astype(jnp.int32)
    tt = jnp.transpose(tensor, (1, 2, 0))
    out_t = _build_gather(B, N, K, D, KP)(tt, idx32)
    return jnp.transpose(out_t, (2, 0, 1))
